# baseline (speedup 1.0000x reference)
import functools

import jax
import jax.numpy as jnp
from jax import lax
from jax.experimental import pallas as pl
from jax.experimental.pallas import tpu as pltpu
from jax.experimental.pallas import tpu_sc as plsc

NUM_CORES = 2
NUM_SUBCORES = 16
NUM_WORKERS = NUM_CORES * NUM_SUBCORES
LANES = 16
CHUNK = 128


def _make_gather(batch, dim):
    assert batch % (NUM_WORKERS * CHUNK) == 0 and dim % LANES == 0
    b_per_w = batch // NUM_WORKERS
    n_chunks = b_per_w // CHUNK
    col_groups = dim // LANES
    mesh = plsc.VectorSubcoreMesh(core_axis_name="c", subcore_axis_name="s")

    @functools.partial(
        pl.kernel,
        mesh=mesh,
        out_type=jax.ShapeDtypeStruct((batch, dim), jnp.float32),
        scratch_types=[
            pltpu.VMEM((b_per_w,), jnp.int32),
            pltpu.VMEM((n_chunks, CHUNK), jnp.int32),
            pltpu.VMEM((CHUNK, 2 * dim), jnp.float32),
            pltpu.VMEM((b_per_w, dim), jnp.float32),
            pltpu.SemaphoreType.DMA,
        ],
    )
    def k(table_hbm, idx_hbm, out_hbm, idx_v, tid_v, stage_v, rows_v, sem):
        wid = lax.axis_index("s") * NUM_CORES + lax.axis_index("c")
        base = wid * b_per_w
        pltpu.sync_copy(idx_hbm.at[pl.ds(base, b_per_w)], idx_v)

        def to_row_ids(g, carry):
            v = idx_v[pl.ds(g * LANES, LANES)]
            tid_v[g // (CHUNK // LANES),
                  pl.ds((g % (CHUNK // LANES)) * LANES, LANES)] = v >> 1
            return carry

        lax.fori_loop(0, b_per_w // LANES, to_row_ids, 0)

        def do_chunk(ch, carry):
            pltpu.async_copy(table_hbm.at[tid_v.at[ch]], stage_v, sem).wait()

            def select(g, c2):
                v = idx_v[pl.ds(ch * CHUNK + g * LANES, LANES)]
                for j in range(LANES):
                    half = (v[j] & 1) * dim
                    p = g * LANES + j
                    for cg in range(col_groups):
                        rows_v[ch * CHUNK + p, pl.ds(cg * LANES, LANES)] = (
                            stage_v[p, pl.ds(half + cg * LANES, LANES)]
                        )
                return c2

            lax.fori_loop(0, CHUNK // LANES, select, 0)
            return carry

        lax.fori_loop(0, n_chunks, do_chunk, 0)
        pltpu.sync_copy(rows_v, out_hbm.at[pl.ds(base, b_per_w)])

    return k


def kernel(idx, vectors):
    batch = idx.shape[0]
    rows, dim = vectors.shape
    table_packed = vectors.reshape(rows // 2, 2 * dim)
    gather = _make_gather(batch, dim)
    return gather(table_packed, idx.astype(jnp.int32))
